# Initial kernel scaffold; baseline (speedup 1.0000x reference)
import jax
import jax.numpy as jnp
from jax.experimental import pallas as pl


def kernel(x, edge_index, lin_W0, lin_b0, agg_W0, agg_b0, lin_W1, lin_b1, agg_W1, agg_b1, post_W1, post_b1, post_W2, post_b2):
    raise NotImplementedError("write your pallas kernel here")


# speedup vs baseline: 3.0008x; 3.0008x over previous
import functools

import jax
import jax.numpy as jnp
from jax import lax
from jax.experimental import pallas as pl
from jax.experimental.pallas import tpu as pltpu
from jax.experimental.pallas import tpu_sc as plsc

N = 10000
D = 128
O_DIM = 40
E = 320000
NW = 32
EPB = 128
NB = 80
E_PAD = NW * NB * EPB
NP = 10112
RPT = NP // 16
BLK = 1000
GRID = N // BLK

_HIGH = jax.lax.Precision.HIGHEST


def _dot(a, b):
    return jnp.dot(a, b, precision=_HIGH, preferred_element_type=jnp.float32)


def _sc_mesh():
    return plsc.VectorSubcoreMesh(core_axis_name="c", subcore_axis_name="s",
                                  num_cores=2, num_subcores=16)


@functools.lru_cache(maxsize=None)
def _make_seg_scatter():
    @functools.partial(
        pl.kernel,
        out_type=jax.ShapeDtypeStruct((2, NP, D), jnp.float32),
        mesh=_sc_mesh(),
        scratch_types=[
            pltpu.VMEM((NB, EPB), jnp.int32),
            pltpu.VMEM((EPB,), jnp.int32),
            pltpu.VMEM((EPB,), jnp.int32),
            pltpu.VMEM((EPB, D), jnp.float32),
            pltpu.VMEM((EPB, D), jnp.float32),
            pltpu.VMEM_SHARED((NP, D), jnp.float32),
            pltpu.SemaphoreType.DMA,
            pltpu.SemaphoreType.DMA,
            pltpu.SemaphoreType.DMA,
            pltpu.SemaphoreType.DMA,
        ],
        compiler_params=pltpu.CompilerParams(use_tc_tiling_on_sc=False),
    )
    def seg(table, src3, dst3, zeros, out,
            src_v, dst_b0, dst_b1, rows0, rows1, accum, g0, g1, s0, s1):
        c = lax.axis_index("c")
        s = lax.axis_index("s")
        wid = c * 16 + s
        pltpu.sync_copy(zeros.at[pl.ds(s * RPT, RPT)],
                        accum.at[pl.ds(s * RPT, RPT)])
        pltpu.sync_copy(src3.at[wid], src_v)
        plsc.subcore_barrier()

        @pl.loop(0, NB, step=2)
        def _(j):
            pltpu.sync_copy(dst3.at[wid, j], dst_b0)
            pltpu.sync_copy(dst3.at[wid, j + 1], dst_b1)
            ga = pltpu.async_copy(table.at[src_v.at[j]], rows0, g0)
            gb = pltpu.async_copy(table.at[src_v.at[j + 1]], rows1, g1)
            ga.wait()
            sa = pltpu.async_copy(rows0, accum.at[dst_b0], s0, add=True)
            gb.wait()
            sb = pltpu.async_copy(rows1, accum.at[dst_b1], s1, add=True)
            sa.wait()
            sb.wait()

        plsc.subcore_barrier()
        pltpu.sync_copy(accum.at[pl.ds(s * RPT, RPT)],
                        out.at[c, pl.ds(s * RPT, RPT)])

    return seg


@functools.lru_cache(maxsize=None)
def _make_edge_count():
    @functools.partial(
        pl.kernel,
        out_type=jax.ShapeDtypeStruct((2, NP, 16), jnp.float32),
        mesh=_sc_mesh(),
        scratch_types=[
            pltpu.VMEM((NB, EPB), jnp.int32),
            pltpu.VMEM((EPB, 16), jnp.float32),
            pltpu.VMEM_SHARED((NP, 16), jnp.float32),
            pltpu.SemaphoreType.DMA,
        ],
        compiler_params=pltpu.CompilerParams(use_tc_tiling_on_sc=False),
    )
    def cntk(dst3, zeros16, ones_hbm, out, dst_v, ones_v, accum, s0):
        c = lax.axis_index("c")
        s = lax.axis_index("s")
        wid = c * 16 + s
        pltpu.sync_copy(zeros16.at[pl.ds(s * RPT, RPT)],
                        accum.at[pl.ds(s * RPT, RPT)])
        pltpu.sync_copy(dst3.at[wid], dst_v)
        pltpu.sync_copy(ones_hbm, ones_v)
        plsc.subcore_barrier()

        @pl.loop(0, NB)
        def _(j):
            pltpu.async_copy(ones_v, accum.at[dst_v.at[j]], s0,
                             add=True).wait()

        plsc.subcore_barrier()
        pltpu.sync_copy(accum.at[pl.ds(s * RPT, RPT)],
                        out.at[c, pl.ds(s * RPT, RPT)])

    return cntk


def _tc_a_body(x_ref, w_ref, b_ref, o_ref):
    o_ref[...] = jnp.maximum(_dot(x_ref[...], w_ref[...]) + b_ref[...], 0.0)


def _tc_a(x, w, b):
    return pl.pallas_call(
        _tc_a_body,
        grid=(GRID,),
        in_specs=[
            pl.BlockSpec((BLK, D), lambda i: (i, 0)),
            pl.BlockSpec((D, D), lambda i: (0, 0)),
            pl.BlockSpec((1, D), lambda i: (0, 0)),
        ],
        out_specs=pl.BlockSpec((BLK, D), lambda i: (i, 0)),
        out_shape=jax.ShapeDtypeStruct((N, D), jnp.float32),
    )(x, w, b)


def _tc_b_body(x_ref, p_ref, c_ref, awx_ref, awm_ref, ab_ref, lw_ref, lb_ref,
               h1_ref, t_ref, inv_ref):
    p = p_ref[0] + p_ref[1]
    cnt = (c_ref[0] + c_ref[1])[:, 0:1]
    inv = 1.0 / jnp.maximum(cnt, 1.0)
    mean = p * inv
    h = jnp.maximum(_dot(x_ref[...], awx_ref[...])
                    + _dot(mean, awm_ref[...]) + ab_ref[...], 0.0)
    nrm = jnp.sqrt(jnp.sum(h * h, axis=1, keepdims=True))
    h1 = h / jnp.maximum(nrm, 1e-12)
    h1_ref[...] = h1
    t_ref[...] = jnp.maximum(_dot(h1, lw_ref[...]) + lb_ref[...], 0.0)
    inv_ref[...] = jnp.broadcast_to(inv, (BLK, D))


def _tc_b(x, partials, cnt_partials, awx, awm, ab, lw, lb):
    return pl.pallas_call(
        _tc_b_body,
        grid=(GRID,),
        in_specs=[
            pl.BlockSpec((BLK, D), lambda i: (i, 0)),
            pl.BlockSpec((2, BLK, D), lambda i: (0, i, 0)),
            pl.BlockSpec((2, BLK, 16), lambda i: (0, i, 0)),
            pl.BlockSpec((D, D), lambda i: (0, 0)),
            pl.BlockSpec((D, D), lambda i: (0, 0)),
            pl.BlockSpec((1, D), lambda i: (0, 0)),
            pl.BlockSpec((D, D), lambda i: (0, 0)),
            pl.BlockSpec((1, D), lambda i: (0, 0)),
        ],
        out_specs=[
            pl.BlockSpec((BLK, D), lambda i: (i, 0)),
            pl.BlockSpec((BLK, D), lambda i: (i, 0)),
            pl.BlockSpec((BLK, D), lambda i: (i, 0)),
        ],
        out_shape=[
            jax.ShapeDtypeStruct((N, D), jnp.float32),
            jax.ShapeDtypeStruct((N, D), jnp.float32),
            jax.ShapeDtypeStruct((N, D), jnp.float32),
        ],
    )(x, partials, cnt_partials, awx, awm, ab, lw, lb)


def _tc_c_body(h1_ref, p_ref, inv_ref, awx_ref, awm_ref, ab_ref,
               pw1_ref, pb1_ref, pw2_ref, pb2_ref, o_ref):
    mean = (p_ref[0] + p_ref[1]) * inv_ref[...]
    h = jnp.maximum(_dot(h1_ref[...], awx_ref[...])
                    + _dot(mean, awm_ref[...]) + ab_ref[...], 0.0)
    nrm = jnp.sqrt(jnp.sum(h * h, axis=1, keepdims=True))
    h2 = h / jnp.maximum(nrm, 1e-12)
    h3 = _dot(h2, pw1_ref[...]) + pb1_ref[...]
    z = _dot(h3, pw2_ref[...]) + pb2_ref[...]
    m = jnp.max(z, axis=1, keepdims=True)
    lse = m + jnp.log(jnp.sum(jnp.exp(z - m), axis=1, keepdims=True))
    o_ref[...] = (z - lse)[:, :O_DIM]


def _tc_c(h1, partials, inv, awx, awm, ab, pw1, pb1, pw2, pb2):
    return pl.pallas_call(
        _tc_c_body,
        grid=(GRID,),
        in_specs=[
            pl.BlockSpec((BLK, D), lambda i: (i, 0)),
            pl.BlockSpec((2, BLK, D), lambda i: (0, i, 0)),
            pl.BlockSpec((BLK, D), lambda i: (i, 0)),
            pl.BlockSpec((D, D), lambda i: (0, 0)),
            pl.BlockSpec((D, D), lambda i: (0, 0)),
            pl.BlockSpec((1, D), lambda i: (0, 0)),
            pl.BlockSpec((D, D), lambda i: (0, 0)),
            pl.BlockSpec((1, D), lambda i: (0, 0)),
            pl.BlockSpec((D, D), lambda i: (0, 0)),
            pl.BlockSpec((1, D), lambda i: (0, 0)),
        ],
        out_specs=pl.BlockSpec((BLK, O_DIM), lambda i: (i, 0)),
        out_shape=jax.ShapeDtypeStruct((N, O_DIM), jnp.float32),
    )(h1, partials, inv, awx, awm, ab, pw1, pb1, pw2, pb2)


def kernel(x, edge_index, lin_W0, lin_b0, agg_W0, agg_b0,
           lin_W1, lin_b1, agg_W1, agg_b1,
           post_W1, post_b1, post_W2, post_b2):
    src = edge_index[0].astype(jnp.int32)
    dst = edge_index[1].astype(jnp.int32)
    pad = E_PAD - E
    src3 = jnp.concatenate([src, jnp.zeros((pad,), jnp.int32)]
                           ).reshape(NW, NB, EPB)
    dst3 = jnp.concatenate([dst, jnp.full((pad,), N, jnp.int32)]
                           ).reshape(NW, NB, EPB)
    zeros128 = jnp.zeros((NP, D), jnp.float32)
    zeros16 = jnp.zeros((NP, 16), jnp.float32)
    ones16 = jnp.ones((EPB, 16), jnp.float32)

    lb0 = lin_b0.reshape(1, D)
    lb1 = lin_b1.reshape(1, D)
    ab0 = agg_b0.reshape(1, D)
    ab1 = agg_b1.reshape(1, D)
    pb1 = post_b1.reshape(1, D)
    pw2 = jnp.pad(post_W2, ((0, 0), (0, D - O_DIM)))
    pb2 = jnp.concatenate([post_b2,
                           jnp.full((D - O_DIM,), -1e30, jnp.float32)]
                          ).reshape(1, D)

    cntp = _make_edge_count()(dst3, zeros16, ones16)
    table0 = _tc_a(x, lin_W0, lb0)
    part0 = _make_seg_scatter()(table0, src3, dst3, zeros128)
    h1, table1, inv = _tc_b(x, part0, cntp, agg_W0[:D], agg_W0[D:], ab0,
                            lin_W1, lb1)
    part1 = _make_seg_scatter()(table1, src3, dst3, zeros128)
    return _tc_c(h1, part1, inv, agg_W1[:D], agg_W1[D:], ab1,
                 post_W1, pb1, pw2, pb2)
